# Initial kernel scaffold; baseline (speedup 1.0000x reference)
import jax
import jax.numpy as jnp
from jax.experimental import pallas as pl


def kernel(x, edge_index, W1, b1, W2, b2):
    raise NotImplementedError("write your pallas kernel here")


# speedup vs baseline: 17.6098x; 17.6098x over previous
import functools

import jax
import jax.numpy as jnp
from jax import lax
from jax.experimental import pallas as pl
from jax.experimental.pallas import tpu as pltpu
from jax.experimental.pallas import tpu_sc as plsc

N = 10000
E = 320000
F = 128
NUM_OUT = 16
NC, NS = 2, 16
NW = NC * NS
EPT = E // NW
CH = 128
NCHUNK = -(-EPT // CH)
EPT_PAD = NCHUNK * CH
DUMMY = N
N_PAD = 10240
SLAB = N_PAD // NS


def _sc_mesh():
    return plsc.VectorSubcoreMesh(
        core_axis_name="c", subcore_axis_name="s", num_cores=NC, num_subcores=NS
    )


def _deg_body(dst_hbm, zeros1_hbm, deg_hbm, dstv, onesv, degsh):
    cid = lax.axis_index("c")
    sid = lax.axis_index("s")
    wid = cid * NS + sid
    pltpu.sync_copy(zeros1_hbm.at[pl.ds(sid * SLAB, SLAB)],
                    degsh.at[pl.ds(sid * SLAB, SLAB)])
    pltpu.sync_copy(dst_hbm.at[wid], dstv)
    for i in range(CH // 16):
        onesv[pl.ds(i * 16, 16)] = jnp.ones((16,), jnp.float32)
    plsc.subcore_barrier()

    def step(j, carry):
        pltpu.sync_copy(onesv, degsh.at[dstv.at[j]], add=True)
        return carry

    lax.fori_loop(0, NCHUNK, step, 0)
    plsc.subcore_barrier()
    pltpu.sync_copy(degsh.at[pl.ds(sid * SLAB, SLAB)],
                    deg_hbm.at[cid, pl.ds(sid * SLAB, SLAB)])


def _sc_degree(dst3, zeros1):
    return pl.kernel(
        _deg_body,
        out_type=jax.ShapeDtypeStruct((NC, N_PAD), jnp.float32),
        mesh=_sc_mesh(),
        scratch_types=[
            pltpu.VMEM((NCHUNK, CH), jnp.int32),
            pltpu.VMEM((CH,), jnp.float32),
            pltpu.VMEM_SHARED((N_PAD,), jnp.float32),
        ],
    )(dst3, zeros1)


def _agg_body(src_hbm, dst_hbm, yp_hbm, norm_hbm, zeros1_hbm, zeros2_hbm,
              raw_hbm, t_hbm, srcv, dstv, rows, nb, rawsh, tsh, sem, sem2):
    cid = lax.axis_index("c")
    sid = lax.axis_index("s")
    wid = cid * NS + sid
    pltpu.sync_copy(zeros2_hbm.at[pl.ds(sid * SLAB, SLAB)],
                    rawsh.at[pl.ds(sid * SLAB, SLAB)])
    pltpu.sync_copy(zeros1_hbm.at[pl.ds(sid * SLAB, SLAB)],
                    tsh.at[pl.ds(sid * SLAB, SLAB)])
    pltpu.sync_copy(src_hbm.at[wid], srcv)
    pltpu.sync_copy(dst_hbm.at[wid], dstv)
    plsc.subcore_barrier()

    def step(j, carry):
        pltpu.async_copy(yp_hbm.at[srcv.at[j]], rows, sem).wait()
        pltpu.sync_copy(rows, rawsh.at[dstv.at[j]], add=True)
        pltpu.async_copy(norm_hbm.at[dstv.at[j]], nb, sem2).wait()
        pltpu.sync_copy(nb, tsh.at[srcv.at[j]], add=True)
        return carry

    lax.fori_loop(0, NCHUNK, step, 0)
    plsc.subcore_barrier()
    pltpu.sync_copy(rawsh.at[pl.ds(sid * SLAB, SLAB)],
                    raw_hbm.at[cid, pl.ds(sid * SLAB, SLAB)])
    pltpu.sync_copy(tsh.at[pl.ds(sid * SLAB, SLAB)],
                    t_hbm.at[cid, pl.ds(sid * SLAB, SLAB)])


def _sc_aggregate(src3, dst3, yp, norm, zeros1, zeros2):
    return pl.kernel(
        _agg_body,
        out_type=(
            jax.ShapeDtypeStruct((NC, N_PAD, F), jnp.float32),
            jax.ShapeDtypeStruct((NC, N_PAD), jnp.float32),
        ),
        mesh=_sc_mesh(),
        scratch_types=[
            pltpu.VMEM((NCHUNK, CH), jnp.int32),
            pltpu.VMEM((NCHUNK, CH), jnp.int32),
            pltpu.VMEM((CH, F), jnp.float32),
            pltpu.VMEM((CH,), jnp.float32),
            pltpu.VMEM_SHARED((N_PAD, F), jnp.float32),
            pltpu.VMEM_SHARED((N_PAD,), jnp.float32),
            pltpu.SemaphoreType.DMA,
            pltpu.SemaphoreType.DMA,
        ],
    )(src3, dst3, yp, norm, zeros1, zeros2)


BLK = 1024


def _prep_body(dega_ref, degb_ref, x_ref, w1_ref, yp_ref, norm_ref):
    deg = dega_ref[...] + degb_ref[...]
    norm = jnp.where(deg > 0.0, lax.rsqrt(deg), 0.0)
    y = jnp.dot(x_ref[...], w1_ref[...], preferred_element_type=jnp.float32)
    yp_ref[...] = y * norm
    norm_ref[...] = norm


def _tc_prep(dega, degb, x_pad, W1):
    grid = N_PAD // BLK
    return pl.pallas_call(
        _prep_body,
        grid=(grid,),
        in_specs=[
            pl.BlockSpec((BLK, 1), lambda i: (i, 0)),
            pl.BlockSpec((BLK, 1), lambda i: (i, 0)),
            pl.BlockSpec((BLK, F), lambda i: (i, 0)),
            pl.BlockSpec((F, F), lambda i: (0, 0)),
        ],
        out_specs=[
            pl.BlockSpec((BLK, F), lambda i: (i, 0)),
            pl.BlockSpec((BLK, 1), lambda i: (i, 0)),
        ],
        out_shape=[
            jax.ShapeDtypeStruct((N_PAD, F), jnp.float32),
            jax.ShapeDtypeStruct((N_PAD, 1), jnp.float32),
        ],
    )(dega, degb, x_pad, W1)


def _final_body(rawa_ref, rawb_ref, ta_ref, tb_ref, norm_ref, b1_ref,
                w2_ref, b2_ref, out_ref, acc_ref):
    i = pl.program_id(0)
    norm = norm_ref[...]
    h1 = jnp.maximum(norm * (rawa_ref[...] + rawb_ref[...]) + b1_ref[...], 0.0)
    s = norm * (ta_ref[...] + tb_ref[...])
    gid = i * BLK + lax.broadcasted_iota(jnp.int32, (BLK, 1), 0)
    s = jnp.where(gid == DUMMY, 0.0, s)
    contrib = jnp.sum(s * h1, axis=0, keepdims=True)

    @pl.when(i == 0)
    def _():
        acc_ref[...] = jnp.zeros_like(acc_ref)

    acc_ref[...] += contrib

    @pl.when(i == pl.num_programs(0) - 1)
    def _():
        v = acc_ref[...] * (1.0 / N)
        out_ref[...] = (
            jnp.dot(v, w2_ref[...], preferred_element_type=jnp.float32)
            + b2_ref[...]
        )


def _tc_final(rawa, rawb, ta, tb, norm, b1r, W2p, b2p):
    grid = N_PAD // BLK
    return pl.pallas_call(
        _final_body,
        grid=(grid,),
        in_specs=[
            pl.BlockSpec((BLK, F), lambda i: (i, 0)),
            pl.BlockSpec((BLK, F), lambda i: (i, 0)),
            pl.BlockSpec((BLK, 1), lambda i: (i, 0)),
            pl.BlockSpec((BLK, 1), lambda i: (i, 0)),
            pl.BlockSpec((BLK, 1), lambda i: (i, 0)),
            pl.BlockSpec((1, F), lambda i: (0, 0)),
            pl.BlockSpec((F, F), lambda i: (0, 0)),
            pl.BlockSpec((1, F), lambda i: (0, 0)),
        ],
        out_specs=pl.BlockSpec((1, F), lambda i: (0, 0)),
        out_shape=jax.ShapeDtypeStruct((1, F), jnp.float32),
        scratch_shapes=[pltpu.VMEM((1, F), jnp.float32)],
    )(rawa, rawb, ta, tb, norm, b1r, W2p, b2p)


def kernel(x, edge_index, W1, b1, W2, b2):
    src = edge_index[0].astype(jnp.int32)
    dst = edge_index[1].astype(jnp.int32)
    src3 = jnp.pad(src.reshape(NW, EPT), ((0, 0), (0, EPT_PAD - EPT)),
                   constant_values=DUMMY).reshape(NW, NCHUNK, CH)
    dst3 = jnp.pad(dst.reshape(NW, EPT), ((0, 0), (0, EPT_PAD - EPT)),
                   constant_values=DUMMY).reshape(NW, NCHUNK, CH)
    x_pad = jnp.pad(x, ((0, N_PAD - N), (0, 0)))
    zeros1 = jnp.zeros((N_PAD,), jnp.float32)
    zeros2 = jnp.zeros((N_PAD, F), jnp.float32)

    deg = _sc_degree(dst3, zeros1)
    dega = deg[0].reshape(N_PAD, 1)
    degb = deg[1].reshape(N_PAD, 1)

    yp, norm_col = _tc_prep(dega, degb, x_pad, W1)
    norm_flat = norm_col.reshape(N_PAD)

    raw, t = _sc_aggregate(src3, dst3, yp, norm_flat, zeros1, zeros2)

    W2p = jnp.pad(W2, ((0, 0), (0, F - W2.shape[1])))
    b2p = jnp.pad(b2, (0, F - b2.shape[0])).reshape(1, F)
    out = _tc_final(raw[0], raw[1], t[0].reshape(N_PAD, 1),
                    t[1].reshape(N_PAD, 1), norm_col, b1.reshape(1, F),
                    W2p, b2p)
    return out[0, :NUM_OUT]
